# baseline (speedup 1.0000x reference)
import functools

import jax
import jax.numpy as jnp
from jax import lax
from jax.experimental import pallas as pl
from jax.experimental.pallas import tpu as pltpu
from jax.experimental.pallas import tpu_sc as plsc

BATCH = 16384
D_EMB = 64
D_FEAT = 128
NC = 2
NS = 16
NW = NC * NS
B_PER_W = BATCH // NW
CHUNK = 128
N_CHUNKS = B_PER_W // CHUNK

@functools.cache
def _build_sc_gather():
    mesh = plsc.VectorSubcoreMesh(core_axis_name="c", subcore_axis_name="s")

    @functools.partial(
        pl.kernel,
        out_type=(
            jax.ShapeDtypeStruct((BATCH, D_EMB), jnp.float32),
            jax.ShapeDtypeStruct((BATCH, D_EMB), jnp.float32),
            jax.ShapeDtypeStruct((BATCH, D_FEAT), jnp.float32),
            jax.ShapeDtypeStruct((BATCH, D_FEAT), jnp.float32),
        ),
        mesh=mesh,
        compiler_params=pltpu.CompilerParams(use_tc_tiling_on_sc=False),
        scratch_types=[
            pltpu.VMEM((CHUNK,), jnp.int32),
            pltpu.VMEM((CHUNK,), jnp.int32),
            pltpu.VMEM((CHUNK, D_EMB), jnp.float32),
            pltpu.VMEM((CHUNK, D_EMB), jnp.float32),
            pltpu.VMEM((CHUNK, D_FEAT), jnp.float32),
            pltpu.VMEM((CHUNK, D_FEAT), jnp.float32),
            pltpu.SemaphoreType.DMA,
        ],
    )
    def _sc_gather(uid_hbm, iid_hbm, ut_hbm, it_hbm, vf_hbm, tf_hbm,
                   u_out, i_out, v_out, t_out,
                   uidx_v, iidx_v, u_buf, i_buf, v_buf, t_buf, sem):
        wid = lax.axis_index("s") * NC + lax.axis_index("c")
        base = wid * B_PER_W
        for c in range(N_CHUNKS):
            off = base + c * CHUNK
            pltpu.sync_copy(uid_hbm.at[pl.ds(off, CHUNK)], uidx_v)
            pltpu.sync_copy(iid_hbm.at[pl.ds(off, CHUNK)], iidx_v)
            g0 = pltpu.async_copy(ut_hbm.at[uidx_v], u_buf, sem)
            g1 = pltpu.async_copy(it_hbm.at[iidx_v], i_buf, sem)
            g2 = pltpu.async_copy(vf_hbm.at[iidx_v], v_buf, sem)
            g3 = pltpu.async_copy(tf_hbm.at[iidx_v], t_buf, sem)
            g0.wait()
            g1.wait()
            g2.wait()
            g3.wait()
            pltpu.sync_copy(u_buf, u_out.at[pl.ds(off, CHUNK)])
            pltpu.sync_copy(i_buf, i_out.at[pl.ds(off, CHUNK)])
            pltpu.sync_copy(v_buf, v_out.at[pl.ds(off, CHUNK)])
            pltpu.sync_copy(t_buf, t_out.at[pl.ds(off, CHUNK)])

    return _sc_gather


BLK = 1024


def _tc_body(u_ref, i_ref, v_ref, t_ref, wv_ref, wt_ref, wf_ref, bf_ref,
             out_ref):
    wf = wf_ref[...]
    a_vis = lax.dot_general(wf[:, :D_EMB], wv_ref[...],
                            (((1,), (0,)), ((), ())),
                            preferred_element_type=jnp.float32)
    a_txt = lax.dot_general(wf[:, D_EMB:], wt_ref[...],
                            (((1,), (0,)), ((), ())),
                            preferred_element_type=jnp.float32)
    mm = lax.dot_general(v_ref[...], a_vis, (((1,), (1,)), ((), ())),
                         preferred_element_type=jnp.float32)
    mm = mm + lax.dot_general(t_ref[...], a_txt, (((1,), (1,)), ((), ())),
                              preferred_element_type=jnp.float32)
    mm = mm + bf_ref[...]
    out_ref[...] = jnp.sum(u_ref[...] * (i_ref[...] + mm), axis=1)


_tc_score = pl.pallas_call(
    _tc_body,
    grid=(BATCH // BLK,),
    in_specs=[
        pl.BlockSpec((BLK, D_EMB), lambda i: (i, 0)),
        pl.BlockSpec((BLK, D_EMB), lambda i: (i, 0)),
        pl.BlockSpec((BLK, D_FEAT), lambda i: (i, 0)),
        pl.BlockSpec((BLK, D_FEAT), lambda i: (i, 0)),
        pl.BlockSpec((D_EMB, D_FEAT), lambda i: (0, 0)),
        pl.BlockSpec((D_EMB, D_FEAT), lambda i: (0, 0)),
        pl.BlockSpec((D_EMB, 2 * D_EMB), lambda i: (0, 0)),
        pl.BlockSpec((1, D_EMB), lambda i: (0, 0)),
    ],
    out_specs=pl.BlockSpec((BLK,), lambda i: (i,)),
    out_shape=jax.ShapeDtypeStruct((BATCH,), jnp.float32),
)


def kernel(user_ids, item_ids, user_table, item_table, visual_features,
           text_features, W_vis, W_txt, W_fuse, b_fuse):
    uid = user_ids.astype(jnp.int32)
    iid = item_ids.astype(jnp.int32)
    u_g, i_g, v_g, t_g = _build_sc_gather()(uid, iid, user_table, item_table,
                                            visual_features, text_features)
    return _tc_score(u_g, i_g, v_g, t_g, W_vis, W_txt, W_fuse,
                     b_fuse.reshape(1, D_EMB))


# speedup vs baseline: 1.6076x; 1.6076x over previous
import functools

import jax
import jax.numpy as jnp
from jax import lax
from jax.experimental import pallas as pl
from jax.experimental.pallas import tpu as pltpu
from jax.experimental.pallas import tpu_sc as plsc

BATCH = 16384
D_EMB = 64
D_FEAT = 128
N_USERS = 1000000
N_ITEMS = 100000
NC = 2
NS = 16
NW = NC * NS
B_PER_W = BATCH // NW
CH = 64
NCH = B_PER_W // CH
CHF = 128
NCHF = B_PER_W // CHF
L = 16


@functools.cache
def _build_sc_gather():
    mesh = plsc.VectorSubcoreMesh(core_axis_name="c", subcore_axis_name="s")

    @functools.partial(
        pl.kernel,
        out_type=(
            jax.ShapeDtypeStruct((BATCH, D_EMB), jnp.float32),
            jax.ShapeDtypeStruct((BATCH, D_EMB), jnp.float32),
            jax.ShapeDtypeStruct((BATCH, D_FEAT), jnp.float32),
            jax.ShapeDtypeStruct((BATCH, D_FEAT), jnp.float32),
        ),
        mesh=mesh,
        scratch_types=[
            pltpu.VMEM((B_PER_W,), jnp.int32),
            pltpu.VMEM((B_PER_W, D_EMB), jnp.float32),
            pltpu.VMEM((CHF,), jnp.int32),
            pltpu.VMEM((CHF, D_FEAT), jnp.float32),
            pltpu.VMEM((CHF, D_FEAT), jnp.float32),
            pltpu.SemaphoreType.DMA,
        ],
    )
    def _sc_gather(uid_hbm, iid_hbm, ut_hbm, it_hbm, vf_hbm, tf_hbm,
                   u_out, i_out, v_out, t_out,
                   idx_v, rows, fidx, vbuf, tbuf, sem):
        wid = lax.axis_index("s") * NC + lax.axis_index("c")
        base = wid * B_PER_W

        def gather_64wide(ids_hbm, tab_hbm, out_hbm):
            pltpu.sync_copy(ids_hbm.at[pl.ds(base, B_PER_W)], idx_v)

            def fire_group(g, carry):
                ids16 = idx_v[pl.ds(g * L, L)]
                for k in range(L):
                    pltpu.make_async_copy(
                        tab_hbm.at[pl.ds(ids16[k], 1)],
                        rows.at[pl.ds(g * L + k, 1)], sem).start()
                return carry

            lax.fori_loop(0, B_PER_W // L, fire_group, 0)
            pltpu.make_async_copy(
                tab_hbm.at[pl.ds(0, B_PER_W)], rows, sem).wait()
            pltpu.sync_copy(rows, out_hbm.at[pl.ds(base, B_PER_W)])

        gather_64wide(uid_hbm, ut_hbm, u_out)
        gather_64wide(iid_hbm, it_hbm, i_out)

        for c in range(NCHF):
            off = base + c * CHF
            pltpu.sync_copy(iid_hbm.at[pl.ds(off, CHF)], fidx)
            gv = pltpu.async_copy(vf_hbm.at[fidx], vbuf, sem)
            gt = pltpu.async_copy(tf_hbm.at[fidx], tbuf, sem)
            gv.wait()
            gt.wait()
            pltpu.sync_copy(vbuf, v_out.at[pl.ds(off, CHF)])
            pltpu.sync_copy(tbuf, t_out.at[pl.ds(off, CHF)])

    return _sc_gather


BLK = 1024


def _tc_body(u_ref, i_ref, v_ref, t_ref, wv_ref, wt_ref, wf_ref, bf_ref,
             out_ref):
    wf = wf_ref[...]
    a_vis = lax.dot_general(wf[:, :D_EMB], wv_ref[...],
                            (((1,), (0,)), ((), ())),
                            preferred_element_type=jnp.float32)
    a_txt = lax.dot_general(wf[:, D_EMB:], wt_ref[...],
                            (((1,), (0,)), ((), ())),
                            preferred_element_type=jnp.float32)
    mm = lax.dot_general(v_ref[...], a_vis, (((1,), (1,)), ((), ())),
                         preferred_element_type=jnp.float32)
    mm = mm + lax.dot_general(t_ref[...], a_txt, (((1,), (1,)), ((), ())),
                              preferred_element_type=jnp.float32)
    mm = mm + bf_ref[...]
    out_ref[...] = jnp.sum(u_ref[...] * (i_ref[...] + mm), axis=1)


_tc_score = pl.pallas_call(
    _tc_body,
    grid=(BATCH // BLK,),
    in_specs=[
        pl.BlockSpec((BLK, D_EMB), lambda i: (i, 0)),
        pl.BlockSpec((BLK, D_EMB), lambda i: (i, 0)),
        pl.BlockSpec((BLK, D_FEAT), lambda i: (i, 0)),
        pl.BlockSpec((BLK, D_FEAT), lambda i: (i, 0)),
        pl.BlockSpec((D_EMB, D_FEAT), lambda i: (0, 0)),
        pl.BlockSpec((D_EMB, D_FEAT), lambda i: (0, 0)),
        pl.BlockSpec((D_EMB, 2 * D_EMB), lambda i: (0, 0)),
        pl.BlockSpec((1, D_EMB), lambda i: (0, 0)),
    ],
    out_specs=pl.BlockSpec((BLK,), lambda i: (i,)),
    out_shape=jax.ShapeDtypeStruct((BATCH,), jnp.float32),
)


def kernel(user_ids, item_ids, user_table, item_table, visual_features,
           text_features, W_vis, W_txt, W_fuse, b_fuse):
    uid = user_ids.astype(jnp.int32)
    iid = item_ids.astype(jnp.int32)
    u_g, i_g, v_g, t_g = _build_sc_gather()(uid, iid, user_table, item_table,
                                            visual_features, text_features)
    return _tc_score(u_g, i_g, v_g, t_g, W_vis, W_txt, W_fuse,
                     b_fuse.reshape(1, D_EMB))
